# baseline (speedup 1.0000x reference)
import jax
import jax.numpy as jnp
from jax import lax
from jax.experimental import pallas as pl
from jax.experimental.pallas import tpu as pltpu
from jax.experimental.pallas import tpu_sc as plsc

MAXL2 = 4096
TBL = MAXL2 + 1
N_CLASS = 2
REV_T = 64


def _sc_body(table_hbm, out_hbm, table_v, revJ, sem):
    nc = 2
    wid = lax.axis_index("s") * nc + lax.axis_index("c")

    pltpu.sync_copy(table_hbm, table_v)

    iota = lax.iota(jnp.int32, 16)

    def drain_class(_g, carry):
        for _u in range(8):
            pltpu.make_async_copy(
                out_hbm.at[0, :, pl.ds(0, 2), :],
                revJ.at[0, pl.ds(0, 32), :, :],
                sem,
            ).wait()
        return carry

    for q in range(N_CLASS):
        cls = wid * N_CLASS + q
        rp = lax.shift_right_logical(cls, 4)
        m = jnp.bitwise_and(cls, 15)
        r = 2 * rp
        a_m = jnp.bitwise_and(4095 - r - 8 * m, 127)

        def build_body(t, carry, q=q, a_m=a_m):
            s0 = (6143 - 128 * t) - a_m
            for ri in range(2):
                for cc in range(8):
                    idx = jnp.clip((s0 + ri - 16 * cc) - iota, 0, MAXL2)
                    revJ[q, t, ri, pl.ds(16 * cc, 16)] = plsc.load_gather(
                        table_v, [idx]
                    )
            return carry

        lax.fori_loop(0, REV_T, build_body, 0)

        def fire_body(g, carry, q=q, r=r, m=m):
            for u in range(8):
                s = g * 8 + u
                b = 16 * s + m
                t0 = 31 - s
                pltpu.async_copy(
                    revJ.at[q, pl.ds(t0, 32), :, :],
                    out_hbm.at[b, :, pl.ds(r, 2), :],
                    sem,
                )
            return carry

        lax.fori_loop(0, 4, fire_body, 0)

    lax.fori_loop(0, 8, drain_class, 0)


@jax.jit
def _bias_sc(table):
    mesh = plsc.VectorSubcoreMesh(core_axis_name="c", subcore_axis_name="s")
    out4 = pl.kernel(
        _sc_body,
        out_type=jax.ShapeDtypeStruct((512, 32, 8, 128), jnp.float32),
        mesh=mesh,
        compiler_params=pltpu.CompilerParams(needs_layout_passes=False),
        scratch_types=[
            pltpu.VMEM((TBL,), jnp.float32),
            pltpu.VMEM((N_CLASS, REV_T, 2, 128), jnp.float32),
            pltpu.SemaphoreType.DMA,
        ],
    )(table)
    return out4.transpose(0, 2, 1, 3).reshape(MAXL2, MAXL2)


def kernel(rel_embedding, T):
    del T
    return _bias_sc(rel_embedding)


# speedup vs baseline: 1.0288x; 1.0288x over previous
import jax
import jax.numpy as jnp
from jax import lax
from jax.experimental import pallas as pl
from jax.experimental.pallas import tpu as pltpu
from jax.experimental.pallas import tpu_sc as plsc

MAXL2 = 4096
TBL = MAXL2 + 1
N_CLASS = 4
REV_T = 64


def _sc_body(table_hbm, out_hbm, table_v, rev3d, sem):
    nc = 2
    wid = lax.axis_index("s") * nc + lax.axis_index("c")

    pltpu.sync_copy(table_hbm, table_v)

    iota = lax.iota(jnp.int32, 16)

    def drain_class(_g, carry):
        for _u in range(8):
            pltpu.make_async_copy(
                out_hbm.at[0, :, 0, :], rev3d.at[0, pl.ds(0, 32), :], sem
            ).wait()
        return carry

    for q in range(N_CLASS):
        cls = wid * N_CLASS + q
        r = lax.shift_right_logical(cls, 4)
        m = jnp.bitwise_and(cls, 15)
        phi = jnp.bitwise_and(4095 - r - 8 * m, 127)

        def build_body(t, carry, q=q, phi=phi):
            s0 = (6143 - 128 * t) - phi
            for cc in range(8):
                idx = jnp.clip((s0 - 16 * cc) - iota, 0, MAXL2)
                rev3d[q, t, pl.ds(16 * cc, 16)] = plsc.load_gather(table_v, [idx])
            return carry

        lax.fori_loop(0, REV_T, build_body, 0)

        if q > 1:
            lax.fori_loop(0, 4, drain_class, 0)

        def fire_body(g, carry, q=q, r=r, m=m):
            for u in range(8):
                s = g * 8 + u
                b = 16 * s + m
                t0 = 31 - s
                pltpu.async_copy(
                    rev3d.at[q, pl.ds(t0, 32), :],
                    out_hbm.at[b, :, r, :],
                    sem,
                )
            return carry

        lax.fori_loop(0, 4, fire_body, 0)

    lax.fori_loop(0, 8, drain_class, 0)


@jax.jit
def _bias_sc(table):
    mesh = plsc.VectorSubcoreMesh(core_axis_name="c", subcore_axis_name="s")
    out4 = pl.kernel(
        _sc_body,
        out_type=jax.ShapeDtypeStruct((512, 32, 8, 128), jnp.float32),
        mesh=mesh,
        compiler_params=pltpu.CompilerParams(needs_layout_passes=False),
        scratch_types=[
            pltpu.VMEM((TBL,), jnp.float32),
            pltpu.VMEM((N_CLASS, REV_T, 128), jnp.float32),
            pltpu.SemaphoreType.DMA,
        ],
    )(table)
    return out4.transpose(0, 2, 1, 3).reshape(MAXL2, MAXL2)


def kernel(rel_embedding, T):
    del T
    return _bias_sc(rel_embedding)
